# baseline (speedup 1.0000x reference)
import jax
import jax.numpy as jnp
from jax.experimental import pallas as pl
from jax.experimental.pallas import tpu as pltpu

_B, _NV, _NH = 8192, 64, 64
_N = _NV + _NH
_BBLK = 256


def _death_rxn_body(isp_ref, mu_ref, ncovrow_ref, mute_ref, ncovte_ref):
    i_sp = isp_ref[0]

    mu_blk = mu_ref[...]
    r = -ncovrow_ref[...]

    lane = jax.lax.broadcasted_iota(jnp.int32, (_BBLK, _N), 1)
    is_lane = lane == i_sp
    mu_i = jnp.sum(jnp.where(is_lane, mu_blk, 0.0), axis=1, keepdims=True)

    row = jnp.where(is_lane, 2.0 * r + mu_i, r)

    mute_ref[...] = jnp.where(is_lane, -mu_blk, 0.0)

    sub3 = jax.lax.broadcasted_iota(jnp.int32, (_BBLK, _N, _N), 1)
    lane3 = jax.lax.broadcasted_iota(jnp.int32, (_BBLK, _N, _N), 2)
    ncovte_ref[...] = jnp.where(
        sub3 == i_sp,
        row[:, None, :],
        jnp.where(lane3 == i_sp, row[:, :, None], 0.0),
    )


def kernel(mu, ncov, i_sp):
    isp_arr = jnp.asarray(i_sp, jnp.int32).reshape((1,))
    ncov_flat = ncov.reshape(_B, _N * _N)
    grid_spec = pltpu.PrefetchScalarGridSpec(
        num_scalar_prefetch=1,
        grid=(_B // _BBLK,),
        in_specs=[
            pl.BlockSpec((_BBLK, _N), lambda b, isp: (b, 0)),
            pl.BlockSpec((_BBLK, _N), lambda b, isp: (b, isp[0])),
        ],
        out_specs=[
            pl.BlockSpec((_BBLK, _N), lambda b, isp: (b, 0)),
            pl.BlockSpec((_BBLK, _N, _N), lambda b, isp: (b, 0, 0)),
        ],
    )
    muTE, ncovTE = pl.pallas_call(
        _death_rxn_body,
        grid_spec=grid_spec,
        out_shape=[
            jax.ShapeDtypeStruct((_B, _N), jnp.float32),
            jax.ShapeDtypeStruct((_B, _N, _N), jnp.float32),
        ],
        compiler_params=pltpu.CompilerParams(
            dimension_semantics=("parallel",),
            vmem_limit_bytes=56 * 1024 * 1024,
        ),
        name="death_rxn_scatter",
    )(isp_arr, mu, ncov_flat)
    return muTE, ncovTE


# speedup vs baseline: 3.1771x; 3.1771x over previous
import jax
import jax.numpy as jnp
from jax.experimental import pallas as pl
from jax.experimental.pallas import tpu as pltpu

_B, _NV, _NH = 8192, 64, 64
_N = _NV + _NH
_BBLK = 256
_SUB = 4


def _death_rxn_body(isp_ref, mu_ref, ncovband_ref, mute_ref, ncovte_ref):
    i_sp = isp_ref[0]
    sub = jax.lax.rem(i_sp, _SUB)

    mu_blk = mu_ref[...]
    band = ncovband_ref[...]

    sub_iota = jax.lax.broadcasted_iota(jnp.int32, (1, 1, _SUB, 1), 2)
    r = -jnp.sum(jnp.where(sub_iota == sub, band, 0.0), axis=(1, 2))

    lane = jax.lax.broadcasted_iota(jnp.int32, (_BBLK, _N), 1)
    is_lane = lane == i_sp
    mu_i = jnp.sum(jnp.where(is_lane, mu_blk, 0.0), axis=1, keepdims=True)

    row = jnp.where(is_lane, 2.0 * r + mu_i, r)

    mute_ref[...] = jnp.where(is_lane, -mu_blk, 0.0)

    sub3 = jax.lax.broadcasted_iota(jnp.int32, (_BBLK, _N, _N), 1)
    lane3 = jax.lax.broadcasted_iota(jnp.int32, (_BBLK, _N, _N), 2)
    ncovte_ref[...] = jnp.where(
        sub3 == i_sp,
        row[:, None, :],
        jnp.where(lane3 == i_sp, row[:, :, None], 0.0),
    )


def kernel(mu, ncov, i_sp):
    isp_arr = jnp.asarray(i_sp, jnp.int32).reshape((1,))
    ncov_bands = ncov.reshape(_B, _N // _SUB, _SUB, _N)
    grid_spec = pltpu.PrefetchScalarGridSpec(
        num_scalar_prefetch=1,
        grid=(_B // _BBLK,),
        in_specs=[
            pl.BlockSpec((_BBLK, _N), lambda b, isp: (b, 0)),
            pl.BlockSpec((_BBLK, 1, _SUB, _N), lambda b, isp: (b, isp[0] // _SUB, 0, 0)),
        ],
        out_specs=[
            pl.BlockSpec((_BBLK, _N), lambda b, isp: (b, 0)),
            pl.BlockSpec((_BBLK, _N, _N), lambda b, isp: (b, 0, 0)),
        ],
    )
    muTE, ncovTE = pl.pallas_call(
        _death_rxn_body,
        grid_spec=grid_spec,
        out_shape=[
            jax.ShapeDtypeStruct((_B, _N), jnp.float32),
            jax.ShapeDtypeStruct((_B, _N, _N), jnp.float32),
        ],
        compiler_params=pltpu.CompilerParams(
            dimension_semantics=("parallel",),
            vmem_limit_bytes=56 * 1024 * 1024,
        ),
        name="death_rxn_scatter",
    )(isp_arr, mu, ncov_bands)
    return muTE, ncovTE


# speedup vs baseline: 3.2397x; 1.0197x over previous
import jax
import jax.numpy as jnp
from jax.experimental import pallas as pl
from jax.experimental.pallas import tpu as pltpu

_B, _NV, _NH = 8192, 64, 64
_N = _NV + _NH
_BBLK = 256
_SUB = 2


def _death_rxn_body(isp_ref, mu_ref, ncovband_ref, mute_ref, ncovte_ref):
    i_sp = isp_ref[0]
    sub = jax.lax.rem(i_sp, _SUB)

    mu_blk = mu_ref[...]
    band = ncovband_ref[...]

    sub_iota = jax.lax.broadcasted_iota(jnp.int32, (1, 1, _SUB, 1), 2)
    r = -jnp.sum(jnp.where(sub_iota == sub, band, 0.0), axis=(1, 2))

    lane = jax.lax.broadcasted_iota(jnp.int32, (_BBLK, _N), 1)
    is_lane = lane == i_sp
    mu_i = jnp.sum(jnp.where(is_lane, mu_blk, 0.0), axis=1, keepdims=True)

    row = jnp.where(is_lane, 2.0 * r + mu_i, r)

    mute_ref[...] = jnp.where(is_lane, -mu_blk, 0.0)

    sub3 = jax.lax.broadcasted_iota(jnp.int32, (_BBLK, _N, _N), 1)
    lane3 = jax.lax.broadcasted_iota(jnp.int32, (_BBLK, _N, _N), 2)
    ncovte_ref[...] = jnp.where(
        sub3 == i_sp,
        row[:, None, :],
        jnp.where(lane3 == i_sp, row[:, :, None], 0.0),
    )


def kernel(mu, ncov, i_sp):
    isp_arr = jnp.asarray(i_sp, jnp.int32).reshape((1,))
    ncov_bands = ncov.reshape(_B, _N // _SUB, _SUB, _N)
    grid_spec = pltpu.PrefetchScalarGridSpec(
        num_scalar_prefetch=1,
        grid=(_B // _BBLK,),
        in_specs=[
            pl.BlockSpec((_BBLK, _N), lambda b, isp: (b, 0)),
            pl.BlockSpec((_BBLK, 1, _SUB, _N), lambda b, isp: (b, isp[0] // _SUB, 0, 0)),
        ],
        out_specs=[
            pl.BlockSpec((_BBLK, _N), lambda b, isp: (b, 0)),
            pl.BlockSpec((_BBLK, _N, _N), lambda b, isp: (b, 0, 0)),
        ],
    )
    muTE, ncovTE = pl.pallas_call(
        _death_rxn_body,
        grid_spec=grid_spec,
        out_shape=[
            jax.ShapeDtypeStruct((_B, _N), jnp.float32),
            jax.ShapeDtypeStruct((_B, _N, _N), jnp.float32),
        ],
        compiler_params=pltpu.CompilerParams(
            dimension_semantics=("parallel",),
            vmem_limit_bytes=56 * 1024 * 1024,
        ),
        name="death_rxn_scatter",
    )(isp_arr, mu, ncov_bands)
    return muTE, ncovTE


# speedup vs baseline: 3.2604x; 1.0064x over previous
import jax
import jax.numpy as jnp
from jax.experimental import pallas as pl
from jax.experimental.pallas import tpu as pltpu

_B, _NV, _NH = 8192, 64, 64
_N = _NV + _NH
_BBLK = 256
_SUB = 1


def _death_rxn_body(isp_ref, mu_ref, ncovband_ref, mute_ref, ncovte_ref):
    i_sp = isp_ref[0]
    sub = jax.lax.rem(i_sp, _SUB)

    mu_blk = mu_ref[...]
    band = ncovband_ref[...]

    sub_iota = jax.lax.broadcasted_iota(jnp.int32, (1, 1, _SUB, 1), 2)
    r = -jnp.sum(jnp.where(sub_iota == sub, band, 0.0), axis=(1, 2))

    lane = jax.lax.broadcasted_iota(jnp.int32, (_BBLK, _N), 1)
    is_lane = lane == i_sp
    mu_i = jnp.sum(jnp.where(is_lane, mu_blk, 0.0), axis=1, keepdims=True)

    row = jnp.where(is_lane, 2.0 * r + mu_i, r)

    mute_ref[...] = jnp.where(is_lane, -mu_blk, 0.0)

    sub3 = jax.lax.broadcasted_iota(jnp.int32, (_BBLK, _N, _N), 1)
    lane3 = jax.lax.broadcasted_iota(jnp.int32, (_BBLK, _N, _N), 2)
    ncovte_ref[...] = jnp.where(
        sub3 == i_sp,
        row[:, None, :],
        jnp.where(lane3 == i_sp, row[:, :, None], 0.0),
    )


def kernel(mu, ncov, i_sp):
    isp_arr = jnp.asarray(i_sp, jnp.int32).reshape((1,))
    ncov_bands = ncov.reshape(_B, _N // _SUB, _SUB, _N)
    grid_spec = pltpu.PrefetchScalarGridSpec(
        num_scalar_prefetch=1,
        grid=(_B // _BBLK,),
        in_specs=[
            pl.BlockSpec((_BBLK, _N), lambda b, isp: (b, 0)),
            pl.BlockSpec((_BBLK, 1, _SUB, _N), lambda b, isp: (b, isp[0] // _SUB, 0, 0)),
        ],
        out_specs=[
            pl.BlockSpec((_BBLK, _N), lambda b, isp: (b, 0)),
            pl.BlockSpec((_BBLK, _N, _N), lambda b, isp: (b, 0, 0)),
        ],
    )
    muTE, ncovTE = pl.pallas_call(
        _death_rxn_body,
        grid_spec=grid_spec,
        out_shape=[
            jax.ShapeDtypeStruct((_B, _N), jnp.float32),
            jax.ShapeDtypeStruct((_B, _N, _N), jnp.float32),
        ],
        compiler_params=pltpu.CompilerParams(
            dimension_semantics=("parallel",),
            vmem_limit_bytes=56 * 1024 * 1024,
        ),
        name="death_rxn_scatter",
    )(isp_arr, mu, ncov_bands)
    return muTE, ncovTE
